# baseline (speedup 1.0000x reference)
import functools

import jax
import jax.numpy as jnp
from jax import lax
from jax.experimental import pallas as pl
from jax.experimental.pallas import tpu as pltpu
from jax.experimental.pallas import tpu_sc as plsc

B, L = 1024, 20
M, E = 1000, 64
H1, H2 = 256, 128
MP = 1024
BB = 256

NC, NS = 2, 16
NW = NC * NS
NRG, NCG = 4, 8
RW = MP // NRG
CW = B // NCG
_DN0 = (((0,), (0,)), ((), ()))


def _counts_sc_body(idsT_hbm, maskT_hbm, out_hbm, ids_v, mask_v, cnt_v):
    wid = lax.axis_index("s") * NC + lax.axis_index("c")
    rb = (wid % NRG) * RW
    cb = (wid // NRG) * CW
    pltpu.sync_copy(idsT_hbm.at[:, pl.ds(cb, CW)], ids_v)
    pltpu.sync_copy(maskT_hbm.at[:, pl.ds(cb, CW)], mask_v)

    zeros16 = jnp.zeros((16,), jnp.float32)

    def _zero_rows(i, _):
        base = i * 16
        for r in range(16):
            for c in range(0, CW, 16):
                cnt_v[base + r, pl.ds(c, 16)] = zeros16
        return 0

    lax.fori_loop(0, RW // 16, _zero_rows, 0)

    lane = lax.iota(jnp.int32, 16)
    for l in range(L):
        for c in range(0, CW, 16):
            ids16 = ids_v[l, pl.ds(c, 16)]
            rows = ids16 - rb
            sel = (ids16 >= rb) & (rows < RW)
            cols = lane + c
            vals = mask_v[l, pl.ds(c, 16)]
            plsc.addupdate_scatter(cnt_v, [rows, cols], vals, mask=sel)

    pltpu.sync_copy(cnt_v, out_hbm.at[pl.ds(rb, RW), pl.ds(cb, CW)])


def _counts_sc(idsT, maskT):
    mesh = plsc.VectorSubcoreMesh(core_axis_name="c", subcore_axis_name="s")
    return pl.kernel(
        _counts_sc_body,
        out_type=jax.ShapeDtypeStruct((MP, B), jnp.float32),
        mesh=mesh,
        compiler_params=pltpu.CompilerParams(needs_layout_passes=False),
        scratch_types=[
            pltpu.VMEM((L, CW), jnp.int32),
            pltpu.VMEM((L, CW), jnp.float32),
            pltpu.VMEM((RW, CW), jnp.float32),
        ],
    )(idsT, maskT)


def _tc_body(countsT_ref, maskT_ref, tableT_ref, wlinT_ref, blin_ref,
             w1_ref, b1_ref, w2_ref, b2_ref, w3T_ref, b3_ref, out_ref):
    f32 = jnp.float32
    bf16 = jnp.bfloat16
    maskT = maskT_ref[...]
    counts = countsT_ref[...].astype(bf16)
    denom = jnp.clip(jnp.sum(maskT, axis=0, keepdims=True), 1.0, None)

    tabT = jnp.concatenate(
        [tableT_ref[...].astype(bf16), jnp.zeros((E, MP - (M + 1)), bf16)],
        axis=1)
    user_sum = jnp.dot(tabT, counts, preferred_element_type=f32)
    user = user_sum / denom

    wlT = jnp.concatenate(
        [wlinT_ref[...].astype(jnp.bfloat16),
         jnp.zeros((M, MP - (M + 1)), jnp.bfloat16)], axis=1)
    lin = jnp.dot(wlT, counts,
                  preferred_element_type=f32) + blin_ref[...][:, None]

    moviesT = tableT_ref[:, 0:M]
    fm = lax.dot_general(moviesT, user, _DN0,
                         preferred_element_type=f32)

    h = jnp.maximum(
        lax.dot_general(w1_ref[...], user, _DN0, preferred_element_type=f32)
        + b1_ref[...][:, None], 0.0)
    h = jnp.maximum(
        lax.dot_general(w2_ref[...], h, _DN0, preferred_element_type=f32)
        + b2_ref[...][:, None], 0.0)
    mlp = jnp.dot(w3T_ref[...], h,
                  preferred_element_type=f32) + b3_ref[...][:, None]
    out_ref[...] = jax.nn.sigmoid(lin + fm + mlp)


def kernel(ids, mask, table, W_lin, b_lin, W1, b1, W2, b2, W3, b3):
    full = lambda shape: pl.BlockSpec(shape, lambda i: tuple(0 for _ in shape))
    idsT = ids.astype(jnp.int32).T
    maskT = mask.T
    countsT = _counts_sc(idsT, maskT)
    args = (countsT, maskT, table.T, W_lin.T, b_lin, W1, b1, W2, b2, W3.T, b3)
    args = tuple(pltpu.with_memory_space_constraint(x, pltpu.MemorySpace.HBM)
                 for x in args)
    outT = pl.pallas_call(
        _tc_body,
        grid=(B // BB,),
        in_specs=[
            pl.BlockSpec((MP, BB), lambda i: (0, i)),
            pl.BlockSpec((L, BB), lambda i: (0, i)),
            full((E, M + 1)),
            full((M, M + 1)),
            full((M,)),
            full((E, H1)),
            full((H1,)),
            full((H1, H2)),
            full((H2,)),
            full((M, H2)),
            full((M,)),
        ],
        out_specs=pl.BlockSpec((M, BB), lambda i: (0, i)),
        out_shape=jax.ShapeDtypeStruct((M, B), jnp.float32),
    )(*args)
    return outT.T


# speedup vs baseline: 1.0571x; 1.0571x over previous
import functools

import jax
import jax.numpy as jnp
from jax import lax
from jax.experimental import pallas as pl
from jax.experimental.pallas import tpu as pltpu
from jax.experimental.pallas import tpu_sc as plsc

B, L = 1024, 20
M, E = 1000, 64
H1, H2 = 256, 128
MP = 1024
BB = 256

NC, NS = 2, 16
NW = NC * NS
NRG, NCG = 4, 8
RW = MP // NRG
CW = B // NCG
_DN0 = (((0,), (0,)), ((), ()))


def _counts_sc_body(idsT_hbm, maskT_hbm, out_hbm, ids_v, mask_v, cnt_v):
    wid = lax.axis_index("s") * NC + lax.axis_index("c")
    rb = (wid % NRG) * RW
    cb = (wid // NRG) * CW
    pltpu.sync_copy(idsT_hbm.at[:, pl.ds(cb, CW)], ids_v)
    pltpu.sync_copy(maskT_hbm.at[:, pl.ds(cb, CW)], mask_v)

    zeros16 = jnp.zeros((16,), jnp.float32)

    def _zero_rows(i, _):
        base = i * 16
        for r in range(16):
            for c in range(0, CW, 16):
                cnt_v[base + r, pl.ds(c, 16)] = zeros16
        return 0

    lax.fori_loop(0, RW // 16, _zero_rows, 0)

    lane = lax.iota(jnp.int32, 16)

    def _scatter_l(l, _):
        for c in range(0, CW, 16):
            ids16 = ids_v[l, pl.ds(c, 16)]
            rows = ids16 - rb
            sel = (ids16 >= rb) & (rows < RW)
            cols = lane + c
            vals = mask_v[l, pl.ds(c, 16)]
            plsc.addupdate_scatter(cnt_v, [rows, cols], vals, mask=sel)
        return 0

    lax.fori_loop(0, L, _scatter_l, 0)

    pltpu.sync_copy(cnt_v, out_hbm.at[pl.ds(rb, RW), pl.ds(cb, CW)])


def _counts_sc(idsT, maskT):
    mesh = plsc.VectorSubcoreMesh(core_axis_name="c", subcore_axis_name="s")
    return pl.kernel(
        _counts_sc_body,
        out_type=jax.ShapeDtypeStruct((MP, B), jnp.float32),
        mesh=mesh,
        compiler_params=pltpu.CompilerParams(needs_layout_passes=False),
        scratch_types=[
            pltpu.VMEM((L, CW), jnp.int32),
            pltpu.VMEM((L, CW), jnp.float32),
            pltpu.VMEM((RW, CW), jnp.float32),
        ],
    )(idsT, maskT)


def _tc_body(countsT_ref, maskT_ref, tableT_ref, wlinT_ref, blin_ref,
             w1_ref, b1_ref, w2_ref, b2_ref, w3T_ref, b3_ref, out_ref):
    f32 = jnp.float32
    bf16 = jnp.bfloat16
    maskT = maskT_ref[...]
    counts = countsT_ref[...].astype(bf16)
    denom = jnp.clip(jnp.sum(maskT, axis=0, keepdims=True), 1.0, None)

    tabT = jnp.concatenate(
        [tableT_ref[...].astype(bf16), jnp.zeros((E, MP - (M + 1)), bf16)],
        axis=1)
    user_sum = jnp.dot(tabT, counts, preferred_element_type=f32)
    user = user_sum / denom

    wlT = jnp.concatenate(
        [wlinT_ref[...].astype(jnp.bfloat16),
         jnp.zeros((M, MP - (M + 1)), jnp.bfloat16)], axis=1)
    lin = jnp.dot(wlT, counts,
                  preferred_element_type=f32) + blin_ref[...][:, None]

    moviesT = tableT_ref[:, 0:M]
    fm = lax.dot_general(moviesT, user, _DN0,
                         preferred_element_type=f32)

    h = jnp.maximum(
        lax.dot_general(w1_ref[...], user, _DN0, preferred_element_type=f32)
        + b1_ref[...][:, None], 0.0)
    h = jnp.maximum(
        lax.dot_general(w2_ref[...], h, _DN0, preferred_element_type=f32)
        + b2_ref[...][:, None], 0.0)
    mlp = jnp.dot(w3T_ref[...], h,
                  preferred_element_type=f32) + b3_ref[...][:, None]
    out_ref[...] = jax.nn.sigmoid(lin + fm + mlp)


def kernel(ids, mask, table, W_lin, b_lin, W1, b1, W2, b2, W3, b3):
    full = lambda shape: pl.BlockSpec(shape, lambda i: tuple(0 for _ in shape))
    idsT = ids.astype(jnp.int32).T
    maskT = mask.T
    countsT = _counts_sc(idsT, maskT)
    args = (countsT, maskT, table.T, W_lin.T, b_lin, W1, b1, W2, b2, W3.T, b3)
    args = tuple(pltpu.with_memory_space_constraint(x, pltpu.MemorySpace.HBM)
                 for x in args)
    outT = pl.pallas_call(
        _tc_body,
        grid=(B // BB,),
        in_specs=[
            pl.BlockSpec((MP, BB), lambda i: (0, i)),
            pl.BlockSpec((L, BB), lambda i: (0, i)),
            full((E, M + 1)),
            full((M, M + 1)),
            full((M,)),
            full((E, H1)),
            full((H1,)),
            full((H1, H2)),
            full((H2,)),
            full((M, H2)),
            full((M,)),
        ],
        out_specs=pl.BlockSpec((M, BB), lambda i: (0, i)),
        out_shape=jax.ShapeDtypeStruct((M, B), jnp.float32),
    )(*args)
    return outT.T
